# baseline (speedup 1.0000x reference)
import functools

import jax
import jax.numpy as jnp
from jax import lax
from jax.experimental import pallas as pl
from jax.experimental.pallas import tpu as pltpu
from jax.experimental.pallas import tpu_sc as plsc

T = 2048
D = 1024
F = 2048
E = 8
EP = 128
BT = 256
NSLOT = T * 2 + E * BT
NB = NSLOT // BT
NEG = -1e30

NC = 2
NS = 16
NW = NC * NS
TPW = T // NW
NV = T // 16


def _router_body(x_ref, Wg_ref, bg_ref, e1_ref, e2_ref, g1_ref, g2_ref):
    logitsT = lax.dot_general(
        Wg_ref[...], x_ref[...],
        dimension_numbers=(((0,), (1,)), ((), ())),
        preferred_element_type=jnp.float32) + bg_ref[...]
    rows = [logitsT[e:e + 1, :] for e in range(E)]
    m1 = rows[0]
    for e in range(1, E):
        m1 = jnp.maximum(m1, rows[e])
    e1 = jnp.full((1, T), E, jnp.int32)
    for e in range(E - 1, -1, -1):
        e1 = jnp.where(rows[e] == m1, e, e1)
    rows2 = [jnp.where(e1 == e, NEG, rows[e]) for e in range(E)]
    m2 = rows2[0]
    for e in range(1, E):
        m2 = jnp.maximum(m2, rows2[e])
    e2 = jnp.full((1, T), E, jnp.int32)
    for e in range(E - 1, -1, -1):
        e2 = jnp.where(rows2[e] == m2, e, e2)
    g1 = 1.0 / (1.0 + jnp.exp(m2 - m1))
    e1_ref[...] = e1
    e2_ref[...] = e2
    g1_ref[...] = g1
    g2_ref[...] = 1.0 - g1


def _router(xf, Wgp, bgp):
    return pl.pallas_call(
        _router_body,
        out_shape=(
            jax.ShapeDtypeStruct((1, T), jnp.int32),
            jax.ShapeDtypeStruct((1, T), jnp.int32),
            jax.ShapeDtypeStruct((1, T), jnp.float32),
            jax.ShapeDtypeStruct((1, T), jnp.float32),
        ),
    )(xf, Wgp, bgp)


def _iota16():
    return lax.iota(jnp.int32, 16)


def _splat(scalar):
    return jnp.full((16,), scalar, jnp.int32)


def _dispatch_body(e1_hbm, e2_hbm, x_hbm,
                   pos1_hbm, pos2_hbm, xs_hbm, bexp_hbm, meta_hbm, runof_hbm,
                   e1_v, e2_v, pos1_v, pos2_v, rows_v, misc_v, sem, semx):
    wid = lax.axis_index("s") * NC + lax.axis_index("c")
    w4 = wid * (TPW // 16)
    base = wid * TPW

    cpx = pltpu.make_async_copy(x_hbm.at[pl.ds(base, TPW)], rows_v, semx)
    cpx.start()

    pltpu.sync_copy(e1_hbm.at[0], e1_v)
    pltpu.sync_copy(e2_hbm.at[0], e2_v)

    iota = _iota16()
    zero = jnp.zeros((16,), jnp.int32)

    def load16(ref, i):
        return ref[pl.ds(i * 16, 16)]

    def count_step(i, carry):
        accs1, accs2 = carry
        v1 = load16(e1_v, i)
        v2 = load16(e2_v, i)
        accs1 = tuple(accs1[e] + jnp.where(v1 == e, 1, 0) for e in range(E))
        accs2 = tuple(accs2[e] + jnp.where(v2 == e, 1, 0) for e in range(E))
        return accs1, accs2

    init = (tuple(zero for _ in range(E)), tuple(zero for _ in range(E)))
    pre = lax.fori_loop(0, w4, count_step, init)
    p1 = [jnp.sum(a) for a in pre[0]]
    p2 = [jnp.sum(a) for a in pre[1]]
    tot = lax.fori_loop(w4, NV, count_step, pre)
    t1 = [jnp.sum(a) for a in tot[0]]
    t2 = [jnp.sum(a) for a in tot[1]]

    totv = zero
    for e in range(E):
        totv = totv + jnp.where(iota == e, t1[e] + t2[e], 0)
    pc = ((totv + (BT - 1)) >> 8) << 8
    po_inc = plsc.cumsum(pc)
    po_exc = po_inc - pc

    po_exc_s = [jnp.sum(jnp.where(iota == e, po_exc, 0)) for e in range(E)]
    start1 = [_splat(po_exc_s[e] + p1[e]) for e in range(E)]
    start2 = [_splat(po_exc_s[e] + t1[e] + p2[e]) for e in range(E)]

    def place(src_ref, starts, pos_ref):
        for j in range(TPW // 16):
            v = load16(src_ref, w4 + j)
            pos = zero
            for e in range(E):
                m = v == e
                mi = jnp.where(m, 1, 0)
                c = plsc.cumsum(mi)
                pos = jnp.where(m, starts[e] + c - 1, pos)
                starts[e] = starts[e] + _splat(jnp.sum(mi))
            pos_ref[pl.ds(j * 16, 16)] = pos

    place(e1_v, start1, pos1_v)
    place(e2_v, start2, pos2_v)

    pltpu.sync_copy(pos1_v, pos1_hbm.at[pl.ds(base, TPW)])
    pltpu.sync_copy(pos2_v, pos2_hbm.at[pl.ds(base, TPW)])

    cpx.wait()
    cp1 = pltpu.make_async_copy(rows_v, xs_hbm.at[pos1_v], sem)
    cp1.start()
    cp2 = pltpu.make_async_copy(rows_v, xs_hbm.at[pos2_v], sem)
    cp2.start()
    cp1.wait()
    cp2.wait()

    @pl.when(wid == 0)
    def _():
        nblk = jnp.sum(jnp.where(iota == E - 1, po_inc, 0)) >> 8
        poi = [jnp.sum(jnp.where(iota == e, po_inc, 0)) for e in range(E)]
        pcs = [jnp.sum(jnp.where(iota == e, pc, 0)) for e in range(E)]
        pres = [jnp.where(pcs[e] > 0, 1, 0) for e in range(E)]
        for j in range(2):
            b = iota + j * 16
            cnt = zero
            rof = zero
            for e in range(E):
                seg_done = jnp.where(b * BT >= poi[e], 1, 0)
                cnt = cnt + seg_done
                rof = rof + seg_done * pres[e]
            misc_v[pl.ds(j * 16, 16)] = jnp.minimum(cnt, E - 1)
            misc_v[pl.ds(48 + j * 16, 16)] = rof
        meta_vec = jnp.where(iota == 0, nblk, 0)
        rank = pres[0] * 0
        re_vec = zero
        for e in range(E):
            re_vec = re_vec + jnp.where(iota == 8 + rank, e, 0) * pres[e]
            rank = rank + pres[e]
        meta_vec = meta_vec + jnp.where(iota == 1, rank, 0) + re_vec
        misc_v[pl.ds(32, 16)] = meta_vec
        pltpu.sync_copy(misc_v.at[pl.ds(0, 32)], bexp_hbm)
        pltpu.sync_copy(misc_v.at[pl.ds(32, 16)], meta_hbm)
        pltpu.sync_copy(misc_v.at[pl.ds(48, 32)], runof_hbm)


def _dispatch(e1, e2, xf):
    mesh = plsc.VectorSubcoreMesh(core_axis_name="c", subcore_axis_name="s")
    fn = pl.kernel(
        _dispatch_body,
        out_type=(
            jax.ShapeDtypeStruct((T,), jnp.int32),
            jax.ShapeDtypeStruct((T,), jnp.int32),
            jax.ShapeDtypeStruct((NSLOT, D), jnp.float32),
            jax.ShapeDtypeStruct((32,), jnp.int32),
            jax.ShapeDtypeStruct((16,), jnp.int32),
            jax.ShapeDtypeStruct((32,), jnp.int32),
        ),
        mesh=mesh,
        scratch_types=[
            pltpu.VMEM((T,), jnp.int32),
            pltpu.VMEM((T,), jnp.int32),
            pltpu.VMEM((TPW,), jnp.int32),
            pltpu.VMEM((TPW,), jnp.int32),
            pltpu.VMEM((TPW, D), jnp.float32),
            pltpu.VMEM((80,), jnp.int32),
            pltpu.SemaphoreType.DMA,
            pltpu.SemaphoreType.DMA,
        ],
        compiler_params=pltpu.CompilerParams(needs_layout_passes=False),
    )
    return fn(e1, e2, xf)


def _ffn_body(bexp_ref, meta_ref, runof_ref, xs_ref, W1_ref, b1_ref,
              W2_ref, b2_ref, ys_ref, w1f_ref, w2f_ref, w1b_ref, w2b_ref,
              sem1, sem2):
    b = pl.program_id(0)
    nblk = meta_ref[0]

    @pl.when(b < nblk)
    def _():
        r = runof_ref[b]
        nrun = meta_ref[1]
        p = jnp.bitwise_and(r, 1)

        @pl.when(b == 0)
        def _cold():
            e0 = meta_ref[8]
            pltpu.make_async_copy(W1_ref.at[e0], w1f_ref.at[0], sem1).start()
            pltpu.make_async_copy(W2_ref.at[e0], w2f_ref.at[0], sem2).start()

        first = (b == 0) | (runof_ref[jnp.maximum(b - 1, 0)] != r)

        @pl.when(first)
        def _swap():
            pltpu.make_async_copy(W1_ref.at[0], w1f_ref.at[0], sem1).wait()
            pltpu.make_async_copy(W2_ref.at[0], w2f_ref.at[0], sem2).wait()

            @pl.when(p == 0)
            def _c0():
                w1b_ref[...] = w1f_ref[0].astype(jnp.bfloat16)
                w2b_ref[...] = w2f_ref[0].astype(jnp.bfloat16)

            @pl.when(p == 1)
            def _c1():
                w1b_ref[...] = w1f_ref[1].astype(jnp.bfloat16)
                w2b_ref[...] = w2f_ref[1].astype(jnp.bfloat16)

            @pl.when(r + 1 < nrun)
            def _pre():
                en = meta_ref[9 + r]
                pn = 1 - p
                pltpu.make_async_copy(
                    W1_ref.at[en], w1f_ref.at[pn], sem1).start()
                pltpu.make_async_copy(
                    W2_ref.at[en], w2f_ref.at[pn], sem2).start()

        xb = xs_ref[...].astype(jnp.bfloat16)
        FS = F // 4
        y = None
        for s in range(4):
            h = jnp.dot(xb, w1b_ref[:, s * FS:(s + 1) * FS],
                        preferred_element_type=jnp.float32)
            h = h + b1_ref[0][:, s * FS:(s + 1) * FS]
            g = jax.nn.gelu(h.astype(jnp.bfloat16))
            part = jnp.dot(g, w2b_ref[s * FS:(s + 1) * FS, :],
                           preferred_element_type=jnp.float32)
            y = part if y is None else y + part
        ys_ref[...] = y + b2_ref[0]


def _ffn(bexp, meta, runof, xs, W1, b1, W2, b2):
    def clamped(bexp_ref, meta_ref, b):
        return jnp.minimum(b, meta_ref[0] - 1)

    grid_spec = pltpu.PrefetchScalarGridSpec(
        num_scalar_prefetch=3,
        grid=(NB,),
        in_specs=[
            pl.BlockSpec((BT, D),
                         lambda b, be, me, ro: (clamped(be, me, b), 0)),
            pl.BlockSpec(memory_space=pl.ANY),
            pl.BlockSpec((1, 1, F),
                         lambda b, be, me, ro: (be[clamped(be, me, b)], 0, 0)),
            pl.BlockSpec(memory_space=pl.ANY),
            pl.BlockSpec((1, 1, D),
                         lambda b, be, me, ro: (be[clamped(be, me, b)], 0, 0)),
        ],
        out_specs=pl.BlockSpec((BT, D),
                               lambda b, be, me, ro: (clamped(be, me, b), 0)),
        scratch_shapes=[
            pltpu.VMEM((2, D, F), jnp.float32),
            pltpu.VMEM((2, F, D), jnp.float32),
            pltpu.VMEM((D, F), jnp.bfloat16),
            pltpu.VMEM((F, D), jnp.bfloat16),
            pltpu.SemaphoreType.DMA,
            pltpu.SemaphoreType.DMA,
        ],
    )
    return pl.pallas_call(
        _ffn_body,
        grid_spec=grid_spec,
        out_shape=jax.ShapeDtypeStruct((NSLOT, D), jnp.float32),
        compiler_params=pltpu.CompilerParams(
            dimension_semantics=("arbitrary",)),
    )(bexp, meta, runof, xs, W1, b1, W2, b2)


CHUNK = 16
NR = TPW // CHUNK


def _combine_body(ys_hbm, pos1_hbm, pos2_hbm, g1_hbm, g2_hbm, out_hbm,
                  idx_v, g_v, rows1_v, rows2_v, outb_v,
                  gsem, osem):
    wid = lax.axis_index("s") * NC + lax.axis_index("c")
    iota = _iota16()

    def start_round(h, p):
        base = wid * TPW + h * CHUNK
        pltpu.sync_copy(pos1_hbm.at[pl.ds(base, CHUNK)], idx_v.at[2 * p])
        pltpu.sync_copy(pos2_hbm.at[pl.ds(base, CHUNK)], idx_v.at[2 * p + 1])
        pltpu.sync_copy(g1_hbm.at[0, pl.ds(base, CHUNK)],
                        g_v.at[2 * p, pl.ds(0, CHUNK)])
        pltpu.sync_copy(g2_hbm.at[0, pl.ds(base, CHUNK)],
                        g_v.at[2 * p + 1, pl.ds(0, CHUNK)])
        pltpu.make_async_copy(
            ys_hbm.at[idx_v.at[2 * p]], rows1_v.at[p], gsem[p]).start()
        pltpu.make_async_copy(
            ys_hbm.at[idx_v.at[2 * p + 1]], rows2_v.at[p], gsem[p]).start()

    start_round(0, 0)
    for h in range(NR):
        p = h % 2
        if h + 1 < NR:
            start_round(h + 1, 1 - p)
        pltpu.make_async_copy(
            ys_hbm.at[idx_v.at[2 * p]], rows1_v.at[p], gsem[p]).wait()
        pltpu.make_async_copy(
            ys_hbm.at[idx_v.at[2 * p + 1]], rows2_v.at[p], gsem[p]).wait()
        base = wid * TPW + h * CHUNK
        if h >= 2:
            pltpu.make_async_copy(
                outb_v.at[p], out_hbm.at[pl.ds(base, CHUNK)], osem[p]).wait()

        @plsc.parallel_loop(0, CHUNK, unroll=2)
        def _row(j):
            w1 = g_v[2 * p, pl.ds(j, 16)]
            w2 = g_v[2 * p + 1, pl.ds(j, 16)]
            a1 = plsc.cumsum(jnp.where(iota == 0, w1, 0.0))
            a2 = plsc.cumsum(jnp.where(iota == 0, w2, 0.0))
            for q in range(D // 16):
                r1 = rows1_v[p, j, pl.ds(q * 16, 16)]
                r2 = rows2_v[p, j, pl.ds(q * 16, 16)]
                outb_v[p, j, pl.ds(q * 16, 16)] = a1 * r1 + a2 * r2
        pltpu.make_async_copy(
            outb_v.at[p], out_hbm.at[pl.ds(base, CHUNK)], osem[p]).start()
    for h in (NR - 2, NR - 1):
        p = h % 2
        base = wid * TPW + h * CHUNK
        pltpu.make_async_copy(
            outb_v.at[p], out_hbm.at[pl.ds(base, CHUNK)], osem[p]).wait()


def _combine(ys, pos1, pos2, g1, g2):
    mesh = plsc.VectorSubcoreMesh(core_axis_name="c", subcore_axis_name="s")
    fn = pl.kernel(
        _combine_body,
        out_type=jax.ShapeDtypeStruct((T, D), jnp.float32),
        mesh=mesh,
        scratch_types=[
            pltpu.VMEM((4, CHUNK), jnp.int32),
            pltpu.VMEM((4, CHUNK + 16), jnp.float32),
            pltpu.VMEM((2, CHUNK, D), jnp.float32),
            pltpu.VMEM((2, CHUNK, D), jnp.float32),
            pltpu.VMEM((2, CHUNK, D), jnp.float32),
            (pltpu.SemaphoreType.DMA, pltpu.SemaphoreType.DMA),
            (pltpu.SemaphoreType.DMA, pltpu.SemaphoreType.DMA),
        ],
        compiler_params=pltpu.CompilerParams(needs_layout_passes=False),
    )
    return fn(ys, pos1, pos2, g1, g2)


def kernel(x, Wg, bg, W1, b1, W2, b2):
    B, S, Dm = x.shape
    xf = x.reshape(-1, Dm)
    e1, e2, g1, g2 = _router(xf, Wg, bg.reshape(E, 1))
    pos1, pos2, xs, bexp, meta, runof = _dispatch(e1, e2, xf)
    ys = _ffn(bexp, meta, runof, xs, W1, b1.reshape(E, 1, F), W2,
              b2.reshape(E, 1, D))
    out = _combine(ys, pos1, pos2, g1, g2)
    return out.reshape(B, S, Dm)


# speedup vs baseline: 1.0347x; 1.0347x over previous
import functools

import jax
import jax.numpy as jnp
from jax import lax
from jax.experimental import pallas as pl
from jax.experimental.pallas import tpu as pltpu
from jax.experimental.pallas import tpu_sc as plsc

T = 2048
D = 1024
F = 2048
E = 8
EP = 128
BT = 256
NSLOT = T * 2 + E * BT
NB = NSLOT // BT
NEG = -1e30

NC = 2
NS = 16
NW = NC * NS
TPW = T // NW
NV = T // 16


def _router_body(x_ref, Wg_ref, bg_ref, e1_ref, e2_ref, g1_ref, g2_ref):
    logitsT = lax.dot_general(
        Wg_ref[...], x_ref[...],
        dimension_numbers=(((0,), (1,)), ((), ())),
        preferred_element_type=jnp.float32) + bg_ref[...]
    rows = [logitsT[e:e + 1, :] for e in range(E)]
    m1 = rows[0]
    for e in range(1, E):
        m1 = jnp.maximum(m1, rows[e])
    e1 = jnp.full((1, T), E, jnp.int32)
    for e in range(E - 1, -1, -1):
        e1 = jnp.where(rows[e] == m1, e, e1)
    rows2 = [jnp.where(e1 == e, NEG, rows[e]) for e in range(E)]
    m2 = rows2[0]
    for e in range(1, E):
        m2 = jnp.maximum(m2, rows2[e])
    e2 = jnp.full((1, T), E, jnp.int32)
    for e in range(E - 1, -1, -1):
        e2 = jnp.where(rows2[e] == m2, e, e2)
    g1 = 1.0 / (1.0 + jnp.exp(m2 - m1))
    e1_ref[...] = e1
    e2_ref[...] = e2
    g1_ref[...] = g1
    g2_ref[...] = 1.0 - g1


def _router(xf, Wgp, bgp):
    return pl.pallas_call(
        _router_body,
        out_shape=(
            jax.ShapeDtypeStruct((1, T), jnp.int32),
            jax.ShapeDtypeStruct((1, T), jnp.int32),
            jax.ShapeDtypeStruct((1, T), jnp.float32),
            jax.ShapeDtypeStruct((1, T), jnp.float32),
        ),
    )(xf, Wgp, bgp)


def _iota16():
    return lax.iota(jnp.int32, 16)


def _splat(scalar):
    return jnp.full((16,), scalar, jnp.int32)


def _dispatch_body(e1_hbm, e2_hbm, x_hbm,
                   pos1_hbm, pos2_hbm, xs_hbm, bexp_hbm, meta_hbm, runof_hbm,
                   e1_v, e2_v, pos1_v, pos2_v, rows_v, misc_v, sem, semx):
    wid = lax.axis_index("s") * NC + lax.axis_index("c")
    w4 = wid * (TPW // 16)
    base = wid * TPW

    cpx = pltpu.make_async_copy(x_hbm.at[pl.ds(base, TPW)], rows_v, semx)
    cpx.start()

    pltpu.sync_copy(e1_hbm.at[0], e1_v)
    pltpu.sync_copy(e2_hbm.at[0], e2_v)

    iota = _iota16()
    zero = jnp.zeros((16,), jnp.int32)

    def load16(ref, i):
        return ref[pl.ds(i * 16, 16)]

    def count_step(i, carry):
        accs1, accs2 = carry
        v1 = load16(e1_v, i)
        v2 = load16(e2_v, i)
        accs1 = tuple(accs1[e] + jnp.where(v1 == e, 1, 0) for e in range(E))
        accs2 = tuple(accs2[e] + jnp.where(v2 == e, 1, 0) for e in range(E))
        return accs1, accs2

    init = (tuple(zero for _ in range(E)), tuple(zero for _ in range(E)))
    pre = lax.fori_loop(0, w4, count_step, init)
    p1 = [jnp.sum(a) for a in pre[0]]
    p2 = [jnp.sum(a) for a in pre[1]]
    tot = lax.fori_loop(w4, NV, count_step, pre)
    t1 = [jnp.sum(a) for a in tot[0]]
    t2 = [jnp.sum(a) for a in tot[1]]

    totv = zero
    for e in range(E):
        totv = totv + jnp.where(iota == e, t1[e] + t2[e], 0)
    pc = ((totv + (BT - 1)) >> 8) << 8
    po_inc = plsc.cumsum(pc)
    po_exc = po_inc - pc

    po_exc_s = [jnp.sum(jnp.where(iota == e, po_exc, 0)) for e in range(E)]
    start1 = [_splat(po_exc_s[e] + p1[e]) for e in range(E)]
    start2 = [_splat(po_exc_s[e] + t1[e] + p2[e]) for e in range(E)]

    def place(src_ref, starts, pos_ref):
        for j in range(TPW // 16):
            v = load16(src_ref, w4 + j)
            pos = zero
            for e in range(E):
                m = v == e
                mi = jnp.where(m, 1, 0)
                c = plsc.cumsum(mi)
                pos = jnp.where(m, starts[e] + c - 1, pos)
                starts[e] = starts[e] + _splat(jnp.sum(mi))
            pos_ref[pl.ds(j * 16, 16)] = pos

    place(e1_v, start1, pos1_v)
    place(e2_v, start2, pos2_v)

    pltpu.sync_copy(pos1_v, pos1_hbm.at[pl.ds(base, TPW)])
    pltpu.sync_copy(pos2_v, pos2_hbm.at[pl.ds(base, TPW)])

    cpx.wait()
    cp1 = pltpu.make_async_copy(rows_v, xs_hbm.at[pos1_v], sem)
    cp1.start()
    cp2 = pltpu.make_async_copy(rows_v, xs_hbm.at[pos2_v], sem)
    cp2.start()
    cp1.wait()
    cp2.wait()

    @pl.when(wid == 0)
    def _():
        nblk = jnp.sum(jnp.where(iota == E - 1, po_inc, 0)) >> 8
        poi = [jnp.sum(jnp.where(iota == e, po_inc, 0)) for e in range(E)]
        pcs = [jnp.sum(jnp.where(iota == e, pc, 0)) for e in range(E)]
        pres = [jnp.where(pcs[e] > 0, 1, 0) for e in range(E)]
        for j in range(2):
            b = iota + j * 16
            cnt = zero
            rof = zero
            for e in range(E):
                seg_done = jnp.where(b * BT >= poi[e], 1, 0)
                cnt = cnt + seg_done
                rof = rof + seg_done * pres[e]
            misc_v[pl.ds(j * 16, 16)] = jnp.minimum(cnt, E - 1)
            misc_v[pl.ds(48 + j * 16, 16)] = rof
        meta_vec = jnp.where(iota == 0, nblk, 0)
        rank = pres[0] * 0
        re_vec = zero
        for e in range(E):
            re_vec = re_vec + jnp.where(iota == 8 + rank, e, 0) * pres[e]
            rank = rank + pres[e]
        meta_vec = meta_vec + jnp.where(iota == 1, rank, 0) + re_vec
        misc_v[pl.ds(32, 16)] = meta_vec
        pltpu.sync_copy(misc_v.at[pl.ds(0, 32)], bexp_hbm)
        pltpu.sync_copy(misc_v.at[pl.ds(32, 16)], meta_hbm)
        pltpu.sync_copy(misc_v.at[pl.ds(48, 32)], runof_hbm)


def _dispatch(e1, e2, xf):
    mesh = plsc.VectorSubcoreMesh(core_axis_name="c", subcore_axis_name="s")
    fn = pl.kernel(
        _dispatch_body,
        out_type=(
            jax.ShapeDtypeStruct((T,), jnp.int32),
            jax.ShapeDtypeStruct((T,), jnp.int32),
            jax.ShapeDtypeStruct((NSLOT, D), jnp.float32),
            jax.ShapeDtypeStruct((32,), jnp.int32),
            jax.ShapeDtypeStruct((16,), jnp.int32),
            jax.ShapeDtypeStruct((32,), jnp.int32),
        ),
        mesh=mesh,
        scratch_types=[
            pltpu.VMEM((T,), jnp.int32),
            pltpu.VMEM((T,), jnp.int32),
            pltpu.VMEM((TPW,), jnp.int32),
            pltpu.VMEM((TPW,), jnp.int32),
            pltpu.VMEM((TPW, D), jnp.float32),
            pltpu.VMEM((80,), jnp.int32),
            pltpu.SemaphoreType.DMA,
            pltpu.SemaphoreType.DMA,
        ],
        compiler_params=pltpu.CompilerParams(needs_layout_passes=False),
    )
    return fn(e1, e2, xf)


def _ffn_body(bexp_ref, meta_ref, runof_ref, xs_ref, W1_ref, b1_ref,
              W2_ref, b2_ref, ys_ref, w1f_ref, w2f_ref, w1b_ref, w2b_ref,
              sem1, sem2):
    b = pl.program_id(0)
    nblk = meta_ref[0]

    @pl.when(b < nblk)
    def _():
        r = runof_ref[b]
        nrun = meta_ref[1]
        p = jnp.bitwise_and(r, 1)

        @pl.when(b == 0)
        def _cold():
            e0 = meta_ref[8]
            pltpu.make_async_copy(W1_ref.at[e0], w1f_ref.at[0], sem1).start()
            pltpu.make_async_copy(W2_ref.at[e0], w2f_ref.at[0], sem2).start()

        first = (b == 0) | (runof_ref[jnp.maximum(b - 1, 0)] != r)

        @pl.when(first)
        def _swap():
            pltpu.make_async_copy(W1_ref.at[0], w1f_ref.at[0], sem1).wait()
            pltpu.make_async_copy(W2_ref.at[0], w2f_ref.at[0], sem2).wait()

            @pl.when(p == 0)
            def _c0():
                w1b_ref[...] = w1f_ref[0].astype(jnp.bfloat16)
                w2b_ref[...] = w2f_ref[0].astype(jnp.bfloat16)

            @pl.when(p == 1)
            def _c1():
                w1b_ref[...] = w1f_ref[1].astype(jnp.bfloat16)
                w2b_ref[...] = w2f_ref[1].astype(jnp.bfloat16)

            @pl.when(r + 1 < nrun)
            def _pre():
                en = meta_ref[9 + r]
                pn = 1 - p
                pltpu.make_async_copy(
                    W1_ref.at[en], w1f_ref.at[pn], sem1).start()
                pltpu.make_async_copy(
                    W2_ref.at[en], w2f_ref.at[pn], sem2).start()

        xb = xs_ref[...].astype(jnp.bfloat16)
        FS = F // 4
        y = None
        for s in range(4):
            h = jnp.dot(xb, w1b_ref[:, s * FS:(s + 1) * FS],
                        preferred_element_type=jnp.float32)
            h = h + b1_ref[0][:, s * FS:(s + 1) * FS]
            g = jax.nn.gelu(h.astype(jnp.bfloat16))
            part = jnp.dot(g, w2b_ref[s * FS:(s + 1) * FS, :],
                           preferred_element_type=jnp.float32)
            y = part if y is None else y + part
        ys_ref[...] = y + b2_ref[0]


def _ffn(bexp, meta, runof, xs, W1, b1, W2, b2):
    def clamped(bexp_ref, meta_ref, b):
        return jnp.minimum(b, meta_ref[0] - 1)

    grid_spec = pltpu.PrefetchScalarGridSpec(
        num_scalar_prefetch=3,
        grid=(NB,),
        in_specs=[
            pl.BlockSpec((BT, D),
                         lambda b, be, me, ro: (clamped(be, me, b), 0)),
            pl.BlockSpec(memory_space=pl.ANY),
            pl.BlockSpec((1, 1, F),
                         lambda b, be, me, ro: (be[clamped(be, me, b)], 0, 0)),
            pl.BlockSpec(memory_space=pl.ANY),
            pl.BlockSpec((1, 1, D),
                         lambda b, be, me, ro: (be[clamped(be, me, b)], 0, 0)),
        ],
        out_specs=pl.BlockSpec((BT, D),
                               lambda b, be, me, ro: (clamped(be, me, b), 0)),
        scratch_shapes=[
            pltpu.VMEM((2, D, F), jnp.float32),
            pltpu.VMEM((2, F, D), jnp.float32),
            pltpu.VMEM((D, F), jnp.bfloat16),
            pltpu.VMEM((F, D), jnp.bfloat16),
            pltpu.SemaphoreType.DMA,
            pltpu.SemaphoreType.DMA,
        ],
    )
    return pl.pallas_call(
        _ffn_body,
        grid_spec=grid_spec,
        out_shape=jax.ShapeDtypeStruct((NSLOT, D), jnp.float32),
        compiler_params=pltpu.CompilerParams(
            dimension_semantics=("arbitrary",)),
    )(bexp, meta, runof, xs, W1, b1, W2, b2)


CHUNK = 16
NR = TPW // CHUNK


def _combine_body(ys_hbm, pos1_hbm, pos2_hbm, g1_hbm, g2_hbm, out_hbm,
                  idx_v, g_v, rows1_v, rows2_v, outb_v,
                  gsem, osem):
    wid = lax.axis_index("s") * NC + lax.axis_index("c")
    iota = _iota16()

    def start_round(h, p):
        base = wid * TPW + h * CHUNK
        pltpu.sync_copy(pos1_hbm.at[pl.ds(base, CHUNK)], idx_v.at[2 * p])
        pltpu.sync_copy(pos2_hbm.at[pl.ds(base, CHUNK)], idx_v.at[2 * p + 1])
        pltpu.sync_copy(g1_hbm.at[0, pl.ds(base, CHUNK)],
                        g_v.at[2 * p, pl.ds(0, CHUNK)])
        pltpu.sync_copy(g2_hbm.at[0, pl.ds(base, CHUNK)],
                        g_v.at[2 * p + 1, pl.ds(0, CHUNK)])
        pltpu.make_async_copy(
            ys_hbm.at[idx_v.at[2 * p]], rows1_v.at[p], gsem[p]).start()
        pltpu.make_async_copy(
            ys_hbm.at[idx_v.at[2 * p + 1]], rows2_v.at[p], gsem[p]).start()

    start_round(0, 0)
    for h in range(NR):
        p = h % 2
        if h + 1 < NR:
            start_round(h + 1, 1 - p)
        pltpu.make_async_copy(
            ys_hbm.at[idx_v.at[2 * p]], rows1_v.at[p], gsem[p]).wait()
        pltpu.make_async_copy(
            ys_hbm.at[idx_v.at[2 * p + 1]], rows2_v.at[p], gsem[p]).wait()
        base = wid * TPW + h * CHUNK
        if h >= 2:
            pltpu.make_async_copy(
                outb_v.at[p], out_hbm.at[pl.ds(base, CHUNK)], osem[p]).wait()

        def row(j, _):
            w1 = g_v[2 * p, pl.ds(j, 16)]
            w2 = g_v[2 * p + 1, pl.ds(j, 16)]
            a1 = plsc.cumsum(jnp.where(iota == 0, w1, 0.0))
            a2 = plsc.cumsum(jnp.where(iota == 0, w2, 0.0))
            for q in range(D // 16):
                r1 = rows1_v[p, j, pl.ds(q * 16, 16)]
                r2 = rows2_v[p, j, pl.ds(q * 16, 16)]
                outb_v[p, j, pl.ds(q * 16, 16)] = a1 * r1 + a2 * r2
            return 0

        lax.fori_loop(0, CHUNK, row, 0)
        pltpu.make_async_copy(
            outb_v.at[p], out_hbm.at[pl.ds(base, CHUNK)], osem[p]).start()
    for h in (NR - 2, NR - 1):
        p = h % 2
        base = wid * TPW + h * CHUNK
        pltpu.make_async_copy(
            outb_v.at[p], out_hbm.at[pl.ds(base, CHUNK)], osem[p]).wait()


def _combine(ys, pos1, pos2, g1, g2):
    mesh = plsc.VectorSubcoreMesh(core_axis_name="c", subcore_axis_name="s")
    fn = pl.kernel(
        _combine_body,
        out_type=jax.ShapeDtypeStruct((T, D), jnp.float32),
        mesh=mesh,
        scratch_types=[
            pltpu.VMEM((4, CHUNK), jnp.int32),
            pltpu.VMEM((4, CHUNK + 16), jnp.float32),
            pltpu.VMEM((2, CHUNK, D), jnp.float32),
            pltpu.VMEM((2, CHUNK, D), jnp.float32),
            pltpu.VMEM((2, CHUNK, D), jnp.float32),
            (pltpu.SemaphoreType.DMA, pltpu.SemaphoreType.DMA),
            (pltpu.SemaphoreType.DMA, pltpu.SemaphoreType.DMA),
        ],
        compiler_params=pltpu.CompilerParams(needs_layout_passes=False),
    )
    return fn(ys, pos1, pos2, g1, g2)


def kernel(x, Wg, bg, W1, b1, W2, b2):
    B, S, Dm = x.shape
    xf = x.reshape(-1, Dm)
    e1, e2, g1, g2 = _router(xf, Wg, bg.reshape(E, 1))
    pos1, pos2, xs, bexp, meta, runof = _dispatch(e1, e2, xf)
    ys = _ffn(bexp, meta, runof, xs, W1, b1.reshape(E, 1, F), W2,
              b2.reshape(E, 1, D))
    out = _combine(ys, pos1, pos2, g1, g2)
    return out.reshape(B, S, Dm)
